# baseline (speedup 1.0000x reference)
import dataclasses
import functools

import jax
import jax.numpy as jnp
from jax import lax
from jax.experimental import pallas as pl
from jax.experimental.pallas import tpu as pltpu
from jax.experimental.pallas import tpu_sc as plsc

_N = 2048
_K = 32768
_G = 8

_NC = 2
_NS = 16
_GPC = _G // _NC
_WPG = _NS // _GPC
_RPW = _N // _WPG
_HB = 65536
_QB = _HB // _WPG
_I32MIN = -(2 ** 31)


def _keys16(p):
    u = lax.bitcast_convert_type(p, jnp.int32)
    return u ^ ((u >> 31) | jnp.int32(_I32MIN))


def _splat_i32(x):
    return jnp.full((16,), x, dtype=jnp.int32)


def _sc_select_kernel(s_hbm, t_hbm, out_hbm, s_v, t_v, hist, merged, tmpq,
                      cand_v, vec16, shared_hist, shared_xch):
    c = lax.axis_index("core")
    sid = lax.axis_index("subcore")
    gl = sid // _WPG
    q = sid % _WPG
    g = c * _GPC + gl
    base_sid = gl * _WPG

    zeros16 = jnp.zeros((16,), jnp.int32)
    ones16 = jnp.ones((16,), jnp.int32)
    mones16 = -ones16
    lanes = lax.iota(jnp.int32, 16)

    if True:
        pltpu.sync_copy(s_hbm.at[g, pl.ds(q * _RPW, _RPW)], s_v)
        pltpu.sync_copy(t_hbm.at[g], t_v)

        def zero_hist():
            @pl.loop(0, _HB // 128)
            def _(i):
                for u in range(8):
                    hist[pl.ds(i * 128 + u * 16, 16)] = zeros16

        def scan_pass(shift_bits, low_mask, filt, b_filter):
            bf16 = _splat_i32(b_filter)

            @pl.loop(0, _RPW // 16)
            def _(rc):
                s16 = s_v[pl.ds(rc * 16, 16)]
                for lane in range(16):
                    sv = s16[lane]

                    @pl.loop(0, _N // 64)
                    def _(jc):
                        for u in range(4):
                            t16 = t_v[pl.ds(jc * 64 + u * 16, 16)]
                            ku = _keys16(sv * t16)
                            b = lax.shift_right_logical(ku, shift_bits) & low_mask
                            if filt:
                                msk = lax.shift_right_logical(ku, 16) == bf16
                                plsc.addupdate_scatter(hist, [b], ones16,
                                                       mask=msk)
                            else:
                                plsc.addupdate_scatter(hist, [b], ones16)

            @pl.loop(0, _RPW // 16)
            def _(dc):
                s16 = s_v[pl.ds(dc * 16, 16)]
                t16 = t_v[pl.ds(q * _RPW + dc * 16, 16)]
                ku = _keys16(s16 * t16)
                b = lax.shift_right_logical(ku, shift_bits) & low_mask
                if filt:
                    msk = lax.shift_right_logical(ku, 16) == bf16
                    plsc.addupdate_scatter(hist, [b], mones16, mask=msk)
                else:
                    plsc.addupdate_scatter(hist, [b], mones16)

        def merge_and_find(kneed):
            pltpu.sync_copy(hist, shared_hist.at[c, sid])
            plsc.subcore_barrier()

            qlo = q * _QB
            pltpu.sync_copy(shared_hist.at[c, base_sid, pl.ds(qlo, _QB)],
                            merged)
            for k in range(1, _WPG):
                pltpu.sync_copy(
                    shared_hist.at[c, base_sid + k, pl.ds(qlo, _QB)], tmpq)

                def addk(i, _):
                    merged[pl.ds(i * 16, 16)] = (merged[pl.ds(i * 16, 16)]
                                                 + tmpq[pl.ds(i * 16, 16)])
                    return 0

                lax.fori_loop(0, _QB // 16, addk, 0)

            def tsum(i, acc):
                return acc + merged[pl.ds(i * 16, 16)]

            tq = jnp.sum(lax.fori_loop(0, _QB // 16, tsum, zeros16))

            cand_v[...] = jnp.where(lanes == 0, _splat_i32(tq), zeros16)
            pltpu.sync_copy(cand_v, shared_xch.at[0, c, sid])
            plsc.subcore_barrier()
            s_above = jnp.int32(0)
            for k in range(_WPG):
                pltpu.sync_copy(shared_xch.at[0, c, base_sid + k], cand_v)
                tk = cand_v[0:16][0]
                s_above = s_above + jnp.where(k > q, tk, 0)

            def scan_body(i, carry):
                a_run, found, b_sel, a_sel = carry
                chunk = merged[pl.ds((_QB // 16 - 1 - i) * 16, 16)]
                rev = lax.rev(chunk, (0,))
                cs = plsc.cumsum(rev)
                tot = cs[15]
                cross = (_splat_i32(a_run) + cs) >= _splat_i32(kneed)
                first = jnp.min(jnp.where(cross, lanes, jnp.int32(16)))
                has = first < 16
                prior = cs - rev
                a_here = a_run + jnp.min(
                    jnp.where(cross, prior, jnp.int32(2 ** 31 - 1)))
                b_here = qlo + (_QB // 16 - 1 - i) * 16 + 15 - first
                rec = jnp.logical_and(jnp.logical_not(found), has)
                b_sel = jnp.where(rec, b_here, b_sel)
                a_sel = jnp.where(rec, a_here, a_sel)
                return (a_run + tot, jnp.logical_or(found, has), b_sel, a_sel)

            _, found, b_sel, a_sel = lax.fori_loop(
                0, _QB // 16, scan_body,
                (s_above, jnp.bool_(False), jnp.int32(0), jnp.int32(0)))

            cand = jnp.where(lanes == 0, _splat_i32(found.astype(jnp.int32)),
                             jnp.where(lanes == 1, _splat_i32(b_sel),
                                       jnp.where(lanes == 2, _splat_i32(a_sel),
                                                 zeros16)))
            cand_v[...] = cand
            pltpu.sync_copy(cand_v, shared_xch.at[1, c, sid])
            plsc.subcore_barrier()
            bstar = jnp.int32(0)
            above = jnp.int32(0)
            for k in range(_WPG):
                pltpu.sync_copy(shared_xch.at[1, c, base_sid + k], cand_v)
                ck = cand_v[0:16]
                take = ck[0] > 0
                bstar = jnp.where(take, ck[1], bstar)
                above = jnp.where(take, ck[2], above)
            return bstar, above

        zero_hist()
        scan_pass(16, jnp.int32(0xFFFF), False, jnp.int32(0))
        b1, above1 = merge_and_find(jnp.int32(_K))

        zero_hist()
        scan_pass(0, jnp.int32(0xFFFF), True, b1)
        kneed2 = jnp.int32(_K) - above1
        b2, _ = merge_and_find(kneed2)

        kui = lax.shift_left(b1, 16) | b2
        theta_bits = jnp.where(kui < 0, kui ^ jnp.int32(_I32MIN), ~kui)
        theta = lax.bitcast_convert_type(theta_bits, jnp.float32)

        @pl.when(q == 0)
        def _():
            vec16[...] = jnp.full((16,), theta, dtype=jnp.float32)
            pltpu.sync_copy(vec16, out_hbm.at[g])


def _sc_select(s2, t2):
    mesh = plsc.VectorSubcoreMesh(core_axis_name="core",
                                  subcore_axis_name="subcore")
    cp = pltpu.CompilerParams()
    if "needs_layout_passes" in pltpu.CompilerParams.__dataclass_fields__:
        cp = dataclasses.replace(cp, needs_layout_passes=False)
    kfn = functools.partial(
        pl.kernel,
        out_type=jax.ShapeDtypeStruct((_G, 16), jnp.float32),
        mesh=mesh,
        compiler_params=cp,
        scratch_types=[
            pltpu.VMEM((_RPW,), jnp.float32),
            pltpu.VMEM((_N,), jnp.float32),
            pltpu.VMEM((_HB,), jnp.int32),
            pltpu.VMEM((_QB,), jnp.int32),
            pltpu.VMEM((_QB,), jnp.int32),
            pltpu.VMEM((16,), jnp.int32),
            pltpu.VMEM((16,), jnp.float32),
            pltpu.HBM((_NC, _NS, _HB), jnp.int32),
            pltpu.HBM((2, _NC, _NS, 16), jnp.int32),
        ],
    )(_sc_select_kernel)
    return kfn(s2, t2)


def _mask_kernel(s_ref, t_ref, th_ref, out_ref):
    s = s_ref[0, 0, :]
    t = t_ref[0, 0, :]
    n = s.shape[0]
    theta = th_ref[pl.program_id(0), 0]
    t_row = t[None, :]
    rb = 256
    for b in range(n // rb):
        r0 = b * rb
        s_blk = s[r0:r0 + rb][:, None]
        p = s_blk * t_row
        rows = lax.broadcasted_iota(jnp.int32, (rb, n), 0) + r0
        cols = lax.broadcasted_iota(jnp.int32, (rb, n), 1)
        sel = (p >= theta) & (rows != cols)
        out_ref[0, r0:r0 + rb, :] = sel.astype(jnp.float32)


def kernel(emb_s, emb_t):
    g = emb_s.shape[0]
    s2 = emb_s.reshape(g, _N)
    t2 = emb_t.reshape(g, _N)
    theta = _sc_select(s2, t2)
    return pl.pallas_call(
        _mask_kernel,
        grid=(g,),
        in_specs=[
            pl.BlockSpec((1, 1, _N), lambda i: (i, 0, 0)),
            pl.BlockSpec((1, 1, _N), lambda i: (i, 0, 0)),
            pl.BlockSpec((_G, 16), lambda i: (0, 0),
                         memory_space=pltpu.SMEM),
        ],
        out_specs=pl.BlockSpec((1, _N, _N), lambda i: (i, 0, 0)),
        out_shape=jax.ShapeDtypeStruct((g, _N, _N), jnp.float32),
    )(s2.reshape(g, 1, _N), t2.reshape(g, 1, _N), theta)


# speedup vs baseline: 3.4959x; 3.4959x over previous
import dataclasses
import functools

import jax
import jax.numpy as jnp
from jax import lax
from jax.experimental import pallas as pl
from jax.experimental.pallas import tpu as pltpu
from jax.experimental.pallas import tpu_sc as plsc

_N = 2048
_K = 32768
_G = 8

_NC = 2
_NS = 16
_GPC = _G // _NC
_WPG = _NS // _GPC
_RPW = _N // _WPG
_HB = 65536
_QB = _HB // _WPG
_I32MIN = -(2 ** 31)


def _keys16(p):
    u = lax.bitcast_convert_type(p, jnp.int32)
    return u ^ ((u >> 31) | jnp.int32(_I32MIN))


def _splat_i32(x):
    return jnp.full((16,), x, dtype=jnp.int32)


def _sc_select_kernel(s_hbm, t_hbm, out_hbm, s_v, t_v, hist, merged, tmpq,
                      cand_v, vec16, shared_hist, shared_xch):
    c = lax.axis_index("core")
    sid = lax.axis_index("subcore")
    gl = sid // _WPG
    q = sid % _WPG
    g = c * _GPC + gl
    base_sid = gl * _WPG

    zeros16 = jnp.zeros((16,), jnp.int32)
    ones16 = jnp.ones((16,), jnp.int32)
    mones16 = -ones16
    lanes = lax.iota(jnp.int32, 16)

    if True:
        pltpu.sync_copy(s_hbm.at[g, pl.ds(q * _RPW, _RPW)], s_v)
        pltpu.sync_copy(t_hbm.at[g], t_v)

        def zero_hist():
            @pl.loop(0, _HB // 128)
            def _(i):
                for u in range(8):
                    hist[pl.ds(i * 128 + u * 16, 16)] = zeros16

        def scan_pass(shift_bits, low_mask, filt, b_filter):
            bf16 = _splat_i32(b_filter)

            def bucket_and_scatter(kus, val):
                if filt:
                    bs = [ku & jnp.int32(0xFFFF) for ku in kus]
                    msks = [lax.shift_right_logical(ku, 16) == bf16
                            for ku in kus]
                    for b, msk in zip(bs, msks):
                        plsc.addupdate_scatter(hist, [b], val, mask=msk)
                else:
                    bs = [lax.shift_right_logical(ku, 16) for ku in kus]
                    for b in bs:
                        plsc.addupdate_scatter(hist, [b], val)

            @pl.loop(0, _RPW // 16)
            def _(rc):
                s16 = s_v[pl.ds(rc * 16, 16)]
                for lane in range(16):
                    sv = s16[lane]

                    @pl.loop(0, _N // 128)
                    def _(jc):
                        ts = [t_v[pl.ds(jc * 128 + u * 16, 16)]
                              for u in range(8)]
                        kus = [_keys16(sv * t16) for t16 in ts]
                        bucket_and_scatter(kus, ones16)

            @pl.loop(0, _RPW // 64)
            def _(dc):
                ss = [s_v[pl.ds(dc * 64 + u * 16, 16)] for u in range(4)]
                ts = [t_v[pl.ds(q * _RPW + dc * 64 + u * 16, 16)]
                      for u in range(4)]
                kus = [_keys16(a * b) for a, b in zip(ss, ts)]
                bucket_and_scatter(kus, mones16)

        def merge_and_find(kneed):
            pltpu.sync_copy(hist, shared_hist.at[c, sid])
            plsc.subcore_barrier()

            qlo = q * _QB
            pltpu.sync_copy(shared_hist.at[c, base_sid, pl.ds(qlo, _QB)],
                            merged)
            for k in range(1, _WPG):
                pltpu.sync_copy(
                    shared_hist.at[c, base_sid + k, pl.ds(qlo, _QB)], tmpq)

                def addk(i, _):
                    merged[pl.ds(i * 16, 16)] = (merged[pl.ds(i * 16, 16)]
                                                 + tmpq[pl.ds(i * 16, 16)])
                    return 0

                lax.fori_loop(0, _QB // 16, addk, 0)

            def tsum(i, acc):
                return acc + merged[pl.ds(i * 16, 16)]

            tq = jnp.sum(lax.fori_loop(0, _QB // 16, tsum, zeros16))

            cand_v[...] = jnp.where(lanes == 0, _splat_i32(tq), zeros16)
            pltpu.sync_copy(cand_v, shared_xch.at[0, c, sid])
            plsc.subcore_barrier()
            s_above = jnp.int32(0)
            for k in range(_WPG):
                pltpu.sync_copy(shared_xch.at[0, c, base_sid + k], cand_v)
                tk = cand_v[0:16][0]
                s_above = s_above + jnp.where(k > q, tk, 0)

            def scan_body(i, carry):
                a_run, found, b_sel, a_sel = carry
                chunk = merged[pl.ds((_QB // 16 - 1 - i) * 16, 16)]
                rev = lax.rev(chunk, (0,))
                cs = plsc.cumsum(rev)
                tot = cs[15]
                cross = (_splat_i32(a_run) + cs) >= _splat_i32(kneed)
                first = jnp.min(jnp.where(cross, lanes, jnp.int32(16)))
                has = first < 16
                prior = cs - rev
                a_here = a_run + jnp.min(
                    jnp.where(cross, prior, jnp.int32(2 ** 31 - 1)))
                b_here = qlo + (_QB // 16 - 1 - i) * 16 + 15 - first
                rec = jnp.logical_and(jnp.logical_not(found), has)
                b_sel = jnp.where(rec, b_here, b_sel)
                a_sel = jnp.where(rec, a_here, a_sel)
                return (a_run + tot, jnp.logical_or(found, has), b_sel, a_sel)

            _, found, b_sel, a_sel = lax.fori_loop(
                0, _QB // 16, scan_body,
                (s_above, jnp.bool_(False), jnp.int32(0), jnp.int32(0)))

            cand = jnp.where(lanes == 0, _splat_i32(found.astype(jnp.int32)),
                             jnp.where(lanes == 1, _splat_i32(b_sel),
                                       jnp.where(lanes == 2, _splat_i32(a_sel),
                                                 zeros16)))
            cand_v[...] = cand
            pltpu.sync_copy(cand_v, shared_xch.at[1, c, sid])
            plsc.subcore_barrier()
            bstar = jnp.int32(0)
            above = jnp.int32(0)
            for k in range(_WPG):
                pltpu.sync_copy(shared_xch.at[1, c, base_sid + k], cand_v)
                ck = cand_v[0:16]
                take = ck[0] > 0
                bstar = jnp.where(take, ck[1], bstar)
                above = jnp.where(take, ck[2], above)
            return bstar, above

        zero_hist()
        scan_pass(16, jnp.int32(0xFFFF), False, jnp.int32(0))
        b1, above1 = merge_and_find(jnp.int32(_K))

        zero_hist()
        scan_pass(0, jnp.int32(0xFFFF), True, b1)
        kneed2 = jnp.int32(_K) - above1
        b2, _ = merge_and_find(kneed2)

        kui = lax.shift_left(b1, 16) | b2
        theta_bits = jnp.where(kui < 0, kui ^ jnp.int32(_I32MIN), ~kui)
        theta = lax.bitcast_convert_type(theta_bits, jnp.float32)

        @pl.when(q == 0)
        def _():
            vec16[...] = jnp.full((16,), theta, dtype=jnp.float32)
            pltpu.sync_copy(vec16, out_hbm.at[g])


def _sc_select(s2, t2):
    mesh = plsc.VectorSubcoreMesh(core_axis_name="core",
                                  subcore_axis_name="subcore")
    cp = pltpu.CompilerParams()
    if "needs_layout_passes" in pltpu.CompilerParams.__dataclass_fields__:
        cp = dataclasses.replace(cp, needs_layout_passes=False)
    kfn = functools.partial(
        pl.kernel,
        out_type=jax.ShapeDtypeStruct((_G, 16), jnp.float32),
        mesh=mesh,
        compiler_params=cp,
        scratch_types=[
            pltpu.VMEM((_RPW,), jnp.float32),
            pltpu.VMEM((_N,), jnp.float32),
            pltpu.VMEM((_HB,), jnp.int32),
            pltpu.VMEM((_QB,), jnp.int32),
            pltpu.VMEM((_QB,), jnp.int32),
            pltpu.VMEM((16,), jnp.int32),
            pltpu.VMEM((16,), jnp.float32),
            pltpu.HBM((_NC, _NS, _HB), jnp.int32),
            pltpu.HBM((2, _NC, _NS, 16), jnp.int32),
        ],
    )(_sc_select_kernel)
    return kfn(s2, t2)


def _mask_kernel(s_ref, t_ref, th_ref, out_ref):
    s = s_ref[0, 0, :]
    t = t_ref[0, 0, :]
    n = s.shape[0]
    theta = th_ref[pl.program_id(0), 0]
    t_row = t[None, :]
    rb = 256
    for b in range(n // rb):
        r0 = b * rb
        s_blk = s[r0:r0 + rb][:, None]
        p = s_blk * t_row
        rows = lax.broadcasted_iota(jnp.int32, (rb, n), 0) + r0
        cols = lax.broadcasted_iota(jnp.int32, (rb, n), 1)
        sel = (p >= theta) & (rows != cols)
        out_ref[0, r0:r0 + rb, :] = sel.astype(jnp.float32)


def kernel(emb_s, emb_t):
    g = emb_s.shape[0]
    s2 = emb_s.reshape(g, _N)
    t2 = emb_t.reshape(g, _N)
    theta = _sc_select(s2, t2)
    return pl.pallas_call(
        _mask_kernel,
        grid=(g,),
        in_specs=[
            pl.BlockSpec((1, 1, _N), lambda i: (i, 0, 0)),
            pl.BlockSpec((1, 1, _N), lambda i: (i, 0, 0)),
            pl.BlockSpec((_G, 16), lambda i: (0, 0),
                         memory_space=pltpu.SMEM),
        ],
        out_specs=pl.BlockSpec((1, _N, _N), lambda i: (i, 0, 0)),
        out_shape=jax.ShapeDtypeStruct((g, _N, _N), jnp.float32),
    )(s2.reshape(g, 1, _N), t2.reshape(g, 1, _N), theta)


# speedup vs baseline: 4.2847x; 1.2256x over previous
import dataclasses
import functools

import jax
import jax.numpy as jnp
from jax import lax
from jax.experimental import pallas as pl
from jax.experimental.pallas import tpu as pltpu
from jax.experimental.pallas import tpu_sc as plsc

_N = 2048
_K = 32768
_G = 8

_NC = 2
_NS = 16
_GPC = _G // _NC
_WPG = _NS // _GPC
_RPW = _N // _WPG
_HB = 65536
_QB = _HB // _WPG
_I32MIN = -(2 ** 31)


def _keys16(p):
    u = lax.bitcast_convert_type(p, jnp.int32)
    return u ^ ((u >> 31) | jnp.int32(_I32MIN))


def _splat_i32(x):
    return jnp.full((16,), x, dtype=jnp.int32)


def _sc_select_kernel(s_hbm, t_hbm, out_hbm, s_v, t_v, hist, merged, tmpq,
                      cand_v, vec16, shared_hist, shared_xch):
    c = lax.axis_index("core")
    sid = lax.axis_index("subcore")
    gl = sid // _WPG
    q = sid % _WPG
    g = c * _GPC + gl
    base_sid = gl * _WPG

    zeros16 = jnp.zeros((16,), jnp.int32)
    ones16 = jnp.ones((16,), jnp.int32)
    mones16 = -ones16
    lanes = lax.iota(jnp.int32, 16)

    if True:
        pltpu.sync_copy(s_hbm.at[g, pl.ds(q * _RPW, _RPW)], s_v)
        pltpu.sync_copy(t_hbm.at[g], t_v)

        def zero_hist():
            @pl.loop(0, _HB // 128)
            def _(i):
                for u in range(8):
                    hist[pl.ds(i * 128 + u * 16, 16)] = zeros16

        def scan_pass(shift_bits, low_mask, filt, b_filter):
            bf16 = _splat_i32(b_filter)

            def bucket_and_scatter(kus, val):
                if filt:
                    bs = [ku & jnp.int32(0xFFFF) for ku in kus]
                    msks = [lax.shift_right_logical(ku, 16) == bf16
                            for ku in kus]
                    for b, msk in zip(bs, msks):
                        plsc.addupdate_scatter(hist, [b], val, mask=msk)
                else:
                    bs = [lax.shift_right_logical(ku, 16) for ku in kus]
                    for b in bs:
                        plsc.addupdate_scatter(hist, [b], val)

            @pl.loop(0, _N // 64)
            def _(jc):
                ts = [t_v[pl.ds(jc * 64 + u * 16, 16)] for u in range(4)]

                @pl.loop(0, _RPW // 16)
                def _(rc):
                    s16 = s_v[pl.ds(rc * 16, 16)]
                    for lane in range(16):
                        sv = s16[lane]
                        kus = [_keys16(sv * t16) for t16 in ts]
                        bucket_and_scatter(kus, ones16)

            @pl.loop(0, _RPW // 64)
            def _(dc):
                ss = [s_v[pl.ds(dc * 64 + u * 16, 16)] for u in range(4)]
                ts = [t_v[pl.ds(q * _RPW + dc * 64 + u * 16, 16)]
                      for u in range(4)]
                kus = [_keys16(a * b) for a, b in zip(ss, ts)]
                bucket_and_scatter(kus, mones16)

        def merge_and_find(kneed):
            pltpu.sync_copy(hist, shared_hist.at[c, sid])
            plsc.subcore_barrier()

            qlo = q * _QB
            pltpu.sync_copy(shared_hist.at[c, base_sid, pl.ds(qlo, _QB)],
                            merged)
            for k in range(1, _WPG):
                pltpu.sync_copy(
                    shared_hist.at[c, base_sid + k, pl.ds(qlo, _QB)], tmpq)

                def addk(i, _):
                    merged[pl.ds(i * 16, 16)] = (merged[pl.ds(i * 16, 16)]
                                                 + tmpq[pl.ds(i * 16, 16)])
                    return 0

                lax.fori_loop(0, _QB // 16, addk, 0)

            def tsum(i, acc):
                return acc + merged[pl.ds(i * 16, 16)]

            tq = jnp.sum(lax.fori_loop(0, _QB // 16, tsum, zeros16))

            cand_v[...] = jnp.where(lanes == 0, _splat_i32(tq), zeros16)
            pltpu.sync_copy(cand_v, shared_xch.at[0, c, sid])
            plsc.subcore_barrier()
            s_above = jnp.int32(0)
            for k in range(_WPG):
                pltpu.sync_copy(shared_xch.at[0, c, base_sid + k], cand_v)
                tk = cand_v[0:16][0]
                s_above = s_above + jnp.where(k > q, tk, 0)

            def scan_body(i, carry):
                a_run, found, b_sel, a_sel = carry
                chunk = merged[pl.ds((_QB // 16 - 1 - i) * 16, 16)]
                rev = lax.rev(chunk, (0,))
                cs = plsc.cumsum(rev)
                tot = cs[15]
                cross = (_splat_i32(a_run) + cs) >= _splat_i32(kneed)
                first = jnp.min(jnp.where(cross, lanes, jnp.int32(16)))
                has = first < 16
                prior = cs - rev
                a_here = a_run + jnp.min(
                    jnp.where(cross, prior, jnp.int32(2 ** 31 - 1)))
                b_here = qlo + (_QB // 16 - 1 - i) * 16 + 15 - first
                rec = jnp.logical_and(jnp.logical_not(found), has)
                b_sel = jnp.where(rec, b_here, b_sel)
                a_sel = jnp.where(rec, a_here, a_sel)
                return (a_run + tot, jnp.logical_or(found, has), b_sel, a_sel)

            _, found, b_sel, a_sel = lax.fori_loop(
                0, _QB // 16, scan_body,
                (s_above, jnp.bool_(False), jnp.int32(0), jnp.int32(0)))

            cand = jnp.where(lanes == 0, _splat_i32(found.astype(jnp.int32)),
                             jnp.where(lanes == 1, _splat_i32(b_sel),
                                       jnp.where(lanes == 2, _splat_i32(a_sel),
                                                 zeros16)))
            cand_v[...] = cand
            pltpu.sync_copy(cand_v, shared_xch.at[1, c, sid])
            plsc.subcore_barrier()
            bstar = jnp.int32(0)
            above = jnp.int32(0)
            for k in range(_WPG):
                pltpu.sync_copy(shared_xch.at[1, c, base_sid + k], cand_v)
                ck = cand_v[0:16]
                take = ck[0] > 0
                bstar = jnp.where(take, ck[1], bstar)
                above = jnp.where(take, ck[2], above)
            return bstar, above

        zero_hist()
        scan_pass(16, jnp.int32(0xFFFF), False, jnp.int32(0))
        b1, above1 = merge_and_find(jnp.int32(_K))

        zero_hist()
        scan_pass(0, jnp.int32(0xFFFF), True, b1)
        kneed2 = jnp.int32(_K) - above1
        b2, _ = merge_and_find(kneed2)

        kui = lax.shift_left(b1, 16) | b2
        theta_bits = jnp.where(kui < 0, kui ^ jnp.int32(_I32MIN), ~kui)
        theta = lax.bitcast_convert_type(theta_bits, jnp.float32)

        @pl.when(q == 0)
        def _():
            vec16[...] = jnp.full((16,), theta, dtype=jnp.float32)
            pltpu.sync_copy(vec16, out_hbm.at[g])


def _sc_select(s2, t2):
    mesh = plsc.VectorSubcoreMesh(core_axis_name="core",
                                  subcore_axis_name="subcore")
    cp = pltpu.CompilerParams()
    if "needs_layout_passes" in pltpu.CompilerParams.__dataclass_fields__:
        cp = dataclasses.replace(cp, needs_layout_passes=False)
    kfn = functools.partial(
        pl.kernel,
        out_type=jax.ShapeDtypeStruct((_G, 16), jnp.float32),
        mesh=mesh,
        compiler_params=cp,
        scratch_types=[
            pltpu.VMEM((_RPW,), jnp.float32),
            pltpu.VMEM((_N,), jnp.float32),
            pltpu.VMEM((_HB,), jnp.int32),
            pltpu.VMEM((_QB,), jnp.int32),
            pltpu.VMEM((_QB,), jnp.int32),
            pltpu.VMEM((16,), jnp.int32),
            pltpu.VMEM((16,), jnp.float32),
            pltpu.HBM((_NC, _NS, _HB), jnp.int32),
            pltpu.HBM((2, _NC, _NS, 16), jnp.int32),
        ],
    )(_sc_select_kernel)
    return kfn(s2, t2)


def _mask_kernel(s_ref, t_ref, th_ref, out_ref):
    s = s_ref[0, 0, :]
    t = t_ref[0, 0, :]
    n = s.shape[0]
    theta = th_ref[pl.program_id(0), 0]
    t_row = t[None, :]
    rb = 256
    for b in range(n // rb):
        r0 = b * rb
        s_blk = s[r0:r0 + rb][:, None]
        p = s_blk * t_row
        rows = lax.broadcasted_iota(jnp.int32, (rb, n), 0) + r0
        cols = lax.broadcasted_iota(jnp.int32, (rb, n), 1)
        sel = (p >= theta) & (rows != cols)
        out_ref[0, r0:r0 + rb, :] = sel.astype(jnp.float32)


def kernel(emb_s, emb_t):
    g = emb_s.shape[0]
    s2 = emb_s.reshape(g, _N)
    t2 = emb_t.reshape(g, _N)
    theta = _sc_select(s2, t2)
    return pl.pallas_call(
        _mask_kernel,
        grid=(g,),
        in_specs=[
            pl.BlockSpec((1, 1, _N), lambda i: (i, 0, 0)),
            pl.BlockSpec((1, 1, _N), lambda i: (i, 0, 0)),
            pl.BlockSpec((_G, 16), lambda i: (0, 0),
                         memory_space=pltpu.SMEM),
        ],
        out_specs=pl.BlockSpec((1, _N, _N), lambda i: (i, 0, 0)),
        out_shape=jax.ShapeDtypeStruct((g, _N, _N), jnp.float32),
    )(s2.reshape(g, 1, _N), t2.reshape(g, 1, _N), theta)
